# Initial kernel scaffold; baseline (speedup 1.0000x reference)
import jax
import jax.numpy as jnp
from jax.experimental import pallas as pl


def kernel(x, edge_index, W0, a_src0, a_dst0, b0, W1, a_src1, a_dst1, b1):
    raise NotImplementedError("write your pallas kernel here")


# speedup vs baseline: 64.5685x; 64.5685x over previous
import functools

import jax
import jax.numpy as jnp
from jax import lax
from jax.experimental import pallas as pl
from jax.experimental.pallas import tpu as pltpu
from jax.experimental.pallas import tpu_sc as plsc

N = 10000
E = 320000
D = 128
HID = 128
HEADS = 8
NCLS = 40

K = 80
G = 5
NTILES = 32
NPT = N // 16
ROWS = 1000
F32 = jnp.float32


def _lane_bcast(vec, idx):
    dnums = lax.GatherDimensionNumbers(
        offset_dims=(), collapsed_slice_dims=(0,), start_index_map=(0,))
    return lax.gather(vec, idx[:, None], dnums, slice_sizes=(1,),
                      mode=lax.GatherScatterMode.PROMISE_IN_BOUNDS)


def _tc1_body(x_ref, w_ref, h_ref, ts_ref, td_ref):
    z = jnp.dot(x_ref[...], w_ref[...], preferred_element_type=F32)
    h_ref[...] = z[:, :HID]
    ts_ref[...] = z[:, HID:HID + 16]
    td_ref[...] = z[:, HID + 16:HID + 32]


def _tc1(x, wcat0):
    return pl.pallas_call(
        _tc1_body,
        grid=(N // ROWS,),
        in_specs=[
            pl.BlockSpec((ROWS, D), lambda i: (i, 0)),
            pl.BlockSpec((D, HID + 32), lambda i: (0, 0)),
        ],
        out_specs=[
            pl.BlockSpec((ROWS, HID), lambda i: (i, 0)),
            pl.BlockSpec((ROWS, 16), lambda i: (i, 0)),
            pl.BlockSpec((ROWS, 16), lambda i: (i, 0)),
        ],
        out_shape=[
            jax.ShapeDtypeStruct((N, HID), F32),
            jax.ShapeDtypeStruct((N, 16), F32),
            jax.ShapeDtypeStruct((N, 16), F32),
        ],
    )(x, wcat0)


def _sc_edge_kernel_l0(src2d, dst2d, ts, td, h0, z16, z128):
    ept = E // NTILES
    nchunks = ept // K
    mesh = plsc.VectorSubcoreMesh(core_axis_name="c", subcore_axis_name="s")

    @functools.partial(
        pl.kernel,
        out_type=(jax.ShapeDtypeStruct((2, N, 16), F32),
                  jax.ShapeDtypeStruct((2, N, HID), F32)),
        mesh=mesh,
        compiler_params=pltpu.CompilerParams(use_tc_tiling_on_sc=False),
        scratch_types=[
            pltpu.VMEM((G, K), jnp.int32),
            pltpu.VMEM((G, K), jnp.int32),
            pltpu.VMEM((K, 16), F32),
            pltpu.VMEM((K, 16), F32),
            pltpu.VMEM((K, 16), F32),
            pltpu.VMEM((K, HID), F32),
            pltpu.VMEM_SHARED((N, 16), F32),
            pltpu.VMEM_SHARED((N, HID), F32),
            pltpu.SemaphoreType.DMA,
            pltpu.SemaphoreType.DMA,
            pltpu.SemaphoreType.DMA,
        ],
    )
    def k(src_hbm, dst_hbm, ts_hbm, td_hbm, h_hbm, z16_hbm, z128_hbm,
          den_out, msg_out, sidx, didx, attS, attD, exb, hrows,
          denacc, msgacc, sem1, sem2, sem3):
        c = lax.axis_index("c")
        s = lax.axis_index("s")
        tid = c * 16 + s

        nslice = pl.ds(s * NPT, NPT)
        pltpu.sync_copy(z16_hbm.at[nslice], denacc.at[nslice])
        pltpu.sync_copy(z128_hbm.at[nslice], msgacc.at[nslice])

        rbase = tid * nchunks
        plsc.subcore_barrier()

        def chunk(srow, drow):
            cp1 = pltpu.async_copy(ts_hbm.at[srow], attS, sem1)
            cp2 = pltpu.async_copy(td_hbm.at[drow], attD, sem2)
            cp3 = pltpu.async_copy(h_hbm.at[srow], hrows, sem3)
            cp1.wait()
            cp2.wait()

            def edge_ex(e, _):
                sv = attS[e] + attD[e]
                sv = jnp.maximum(sv, 0.2 * sv)
                exb[e] = jnp.exp(sv)
                return 0
            lax.fori_loop(0, K, edge_ex, 0)
            pltpu.sync_copy(exb, denacc.at[drow], add=True)

            cp3.wait()

            def edge_msg(e, _):
                exrow = exb[e]
                for v in range(HID // 16):
                    a = _lane_bcast(exrow, jnp.full((16,), v, jnp.int32))
                    hrows[e, pl.ds(16 * v, 16)] = (
                        hrows[e, pl.ds(16 * v, 16)] * a)
                return 0
            lax.fori_loop(0, K, edge_msg, 0)
            pltpu.sync_copy(hrows, msgacc.at[drow], add=True)

        def group(g, carry):
            gbase = rbase + g * G
            pltpu.sync_copy(src_hbm.at[pl.ds(gbase, G)], sidx)
            pltpu.sync_copy(dst_hbm.at[pl.ds(gbase, G)], didx)
            for gg in range(G):
                chunk(sidx.at[gg], didx.at[gg])
            return carry

        lax.fori_loop(0, nchunks // G, group, 0)
        plsc.subcore_barrier()

        pltpu.sync_copy(denacc.at[nslice], den_out.at[c, nslice])
        pltpu.sync_copy(msgacc.at[nslice], msg_out.at[c, nslice])

    return k(src2d, dst2d, ts, td, h0, z16, z128)


def _sc_edge_kernel_l1(src2d, dst2d, ts, td, h1a, h1b, z16, z160, hidx):
    ept = E // 16
    nchunks = ept // K
    half = HEADS * NCLS // 2
    mesh = plsc.VectorSubcoreMesh(core_axis_name="c", subcore_axis_name="s")

    @functools.partial(
        pl.kernel,
        out_type=(jax.ShapeDtypeStruct((2, N, 16), F32),
                  jax.ShapeDtypeStruct((2, N, half), F32)),
        mesh=mesh,
        compiler_params=pltpu.CompilerParams(use_tc_tiling_on_sc=False),
        scratch_types=[
            pltpu.VMEM((G, K), jnp.int32),
            pltpu.VMEM((G, K), jnp.int32),
            pltpu.VMEM((K, 16), F32),
            pltpu.VMEM((K, 16), F32),
            pltpu.VMEM((K, 16), F32),
            pltpu.VMEM((K, half), F32),
            pltpu.VMEM((half // 16, 16), jnp.int32),
            pltpu.VMEM_SHARED((N, 16), F32),
            pltpu.VMEM_SHARED((N, half), F32),
            pltpu.SemaphoreType.DMA,
            pltpu.SemaphoreType.DMA,
            pltpu.SemaphoreType.DMA,
        ],
    )
    def k(src_hbm, dst_hbm, ts_hbm, td_hbm, ha_hbm, hb_hbm, z16_hbm, z160_hbm,
          hidx_hbm, den_out, msg_out, sidx, didx, attS, attD, exb, hrows,
          hidxv, denacc, msgacc, sem1, sem2, sem3):
        c = lax.axis_index("c")
        s = lax.axis_index("s")

        nslice = pl.ds(s * NPT, NPT)
        pltpu.sync_copy(z16_hbm.at[nslice], denacc.at[nslice])
        pltpu.sync_copy(z160_hbm.at[nslice], msgacc.at[nslice])

        rbase = s * nchunks
        pltpu.sync_copy(hidx_hbm.at[c], hidxv)

        plsc.subcore_barrier()

        def run(h_hbm):
            headidx = [hidxv[v] for v in range(half // 16)]

            def chunk(srow, drow):
                cp1 = pltpu.async_copy(ts_hbm.at[srow], attS, sem1)
                cp2 = pltpu.async_copy(td_hbm.at[drow], attD, sem2)
                cp3 = pltpu.async_copy(h_hbm.at[srow], hrows, sem3)
                cp1.wait()
                cp2.wait()

                def edge_ex(e, _):
                    sv = attS[e] + attD[e]
                    sv = jnp.maximum(sv, 0.2 * sv)
                    exb[e] = jnp.exp(sv)
                    return 0
                lax.fori_loop(0, K, edge_ex, 0)
                pltpu.sync_copy(exb, denacc.at[drow], add=True)

                cp3.wait()

                def edge_msg(e, _):
                    exrow = exb[e]
                    for v in range(half // 16):
                        a = _lane_bcast(exrow, headidx[v])
                        hrows[e, pl.ds(16 * v, 16)] = (
                            hrows[e, pl.ds(16 * v, 16)] * a)
                    return 0
                lax.fori_loop(0, K, edge_msg, 0)
                pltpu.sync_copy(hrows, msgacc.at[drow], add=True)

            def group(g, carry):
                gbase = rbase + g * G
                pltpu.sync_copy(src_hbm.at[pl.ds(gbase, G)], sidx)
                pltpu.sync_copy(dst_hbm.at[pl.ds(gbase, G)], didx)
                for gg in range(G):
                    chunk(sidx.at[gg], didx.at[gg])
                return carry

            lax.fori_loop(0, nchunks // G, group, 0)

        @pl.when(c == 0)
        def _():
            run(ha_hbm)

        @pl.when(c == 1)
        def _():
            run(hb_hbm)

        plsc.subcore_barrier()
        pltpu.sync_copy(denacc.at[nslice], den_out.at[c, nslice])
        pltpu.sync_copy(msgacc.at[nslice], msg_out.at[c, nslice])

    return k(src2d, dst2d, ts, td, h1a, h1b, z16, z160, hidx)


def _tc2_body(m0_ref, m1_ref, d0_ref, d1_ref, b0_ref, p0_ref, w_ref,
              ha_ref, hb_ref, ts_ref, td_ref):
    msg = m0_ref[0] + m1_ref[0]
    den = d0_ref[0] + d1_ref[0]
    rden = 1.0 / (den + 1e-16)
    scale = jnp.dot(rden, p0_ref[...], preferred_element_type=F32)
    h = msg * scale + b0_ref[...]
    h = jnp.where(h > 0, h, jnp.exp(h) - 1.0)
    z = jnp.dot(h, w_ref[...], preferred_element_type=F32)
    half = HEADS * NCLS // 2
    ha_ref[...] = z[:, :half]
    hb_ref[...] = z[:, half:2 * half]
    ts_ref[...] = z[:, 2 * half:2 * half + 16]
    td_ref[...] = z[:, 2 * half + 16:2 * half + 32]


def _tc2(msg0p, den0p, b0, p0, wcat1):
    half = HEADS * NCLS // 2
    wdim = 2 * half + 32
    return pl.pallas_call(
        _tc2_body,
        grid=(N // ROWS,),
        in_specs=[
            pl.BlockSpec((1, ROWS, HID), lambda i: (0, i, 0)),
            pl.BlockSpec((1, ROWS, HID), lambda i: (1, i, 0)),
            pl.BlockSpec((1, ROWS, 16), lambda i: (0, i, 0)),
            pl.BlockSpec((1, ROWS, 16), lambda i: (1, i, 0)),
            pl.BlockSpec((1, HID), lambda i: (0, 0)),
            pl.BlockSpec((16, HID), lambda i: (0, 0)),
            pl.BlockSpec((HID, wdim), lambda i: (0, 0)),
        ],
        out_specs=[
            pl.BlockSpec((ROWS, half), lambda i: (i, 0)),
            pl.BlockSpec((ROWS, half), lambda i: (i, 0)),
            pl.BlockSpec((ROWS, 16), lambda i: (i, 0)),
            pl.BlockSpec((ROWS, 16), lambda i: (i, 0)),
        ],
        out_shape=[
            jax.ShapeDtypeStruct((N, half), F32),
            jax.ShapeDtypeStruct((N, half), F32),
            jax.ShapeDtypeStruct((N, 16), F32),
            jax.ShapeDtypeStruct((N, 16), F32),
        ],
    )(msg0p, msg0p, den0p, den0p, b0, p0, wcat1)


def _tc3_body(ma_ref, mb_ref, d_ref, p1a_ref, p1b_ref, mm_ref, b1_ref,
              out_ref):
    den = d_ref[0]
    rden = 1.0 / (den + 1e-16)
    sa = jnp.dot(rden, p1a_ref[...], preferred_element_type=F32)
    sb = jnp.dot(rden, p1b_ref[...], preferred_element_type=F32)
    za = jnp.dot(ma_ref[0] * sa, mm_ref[...], preferred_element_type=F32)
    zb = jnp.dot(mb_ref[0] * sb, mm_ref[...], preferred_element_type=F32)
    out_ref[...] = za + zb + b1_ref[...]


def _tc3(msg1p, den1p, p1a, p1b, mm, b1):
    half = HEADS * NCLS // 2
    return pl.pallas_call(
        _tc3_body,
        grid=(N // ROWS,),
        in_specs=[
            pl.BlockSpec((1, ROWS, half), lambda i: (0, i, 0)),
            pl.BlockSpec((1, ROWS, half), lambda i: (1, i, 0)),
            pl.BlockSpec((1, ROWS, 16), lambda i: (0, i, 0)),
            pl.BlockSpec((16, half), lambda i: (0, 0)),
            pl.BlockSpec((16, half), lambda i: (0, 0)),
            pl.BlockSpec((half, NCLS), lambda i: (0, 0)),
            pl.BlockSpec((1, NCLS), lambda i: (0, 0)),
        ],
        out_specs=pl.BlockSpec((ROWS, NCLS), lambda i: (i, 0)),
        out_shape=jax.ShapeDtypeStruct((N, NCLS), F32),
    )(msg1p, msg1p, den1p, p1a, p1b, mm, b1)


def _fold(W, a_src, a_dst, heads):
    d = W.shape[0]
    c = W.shape[1] // heads
    wr = W.reshape(d, heads, c)
    vs = jnp.einsum('dhc,hc->dh', wr, a_src)
    vd = jnp.einsum('dhc,hc->dh', wr, a_dst)
    pad = jnp.zeros((d, 8), F32)
    return jnp.concatenate([vs, pad], 1), jnp.concatenate([vd, pad], 1)


def kernel(x, edge_index, W0, a_src0, a_dst0, b0, W1, a_src1, a_dst1, b1):
    half = HEADS * NCLS // 2
    vs0, vd0 = _fold(W0, a_src0, a_dst0, HEADS)
    wcat0 = jnp.concatenate([W0, vs0, vd0], 1)
    vs1, vd1 = _fold(W1, a_src1, a_dst1, HEADS)
    wcat1 = jnp.concatenate([W1, vs1, vd1], 1)
    p0 = jnp.concatenate(
        [jnp.kron(jnp.eye(HEADS, dtype=F32), jnp.ones((1, HID // HEADS), F32)),
         jnp.zeros((8, HID), F32)], 0)
    blk = jnp.kron(jnp.eye(4, dtype=F32), jnp.ones((1, NCLS), F32))
    p1a = jnp.concatenate([blk, jnp.zeros((12, half), F32)], 0)
    p1b = jnp.concatenate([jnp.zeros((4, half), F32), blk,
                           jnp.zeros((8, half), F32)], 0)
    mm = jnp.tile(jnp.eye(NCLS, dtype=F32), (4, 1)) / HEADS
    src2d = edge_index[0].astype(jnp.int32).reshape(E // K, K)
    dst2d = edge_index[1].astype(jnp.int32).reshape(E // K, K)
    z16 = jnp.zeros((N, 16), F32)
    z128 = jnp.zeros((N, HID), F32)
    z160 = jnp.zeros((N, half), F32)
    hidx = jnp.array([[[(c * half + 16 * v + l) // NCLS for l in range(16)]
                       for v in range(half // 16)]
                      for c in range(2)], jnp.int32)

    h0, ts0, td0 = _tc1(x, wcat0)
    den0p, msg0p = _sc_edge_kernel_l0(src2d, dst2d, ts0, td0, h0, z16, z128)
    h1a, h1b, ts1, td1 = _tc2(msg0p, den0p, b0.reshape(1, HID), p0, wcat1)
    den1p, msg1p = _sc_edge_kernel_l1(src2d, dst2d, ts1, td1, h1a, h1b,
                                      z16, z160, hidx)
    out = _tc3(msg1p, den1p, p1a, p1b, mm, b1.reshape(1, NCLS))
    return out
